# baseline (speedup 1.0000x reference)
import functools
import math

import jax
import jax.numpy as jnp
from jax.experimental import pallas as pl
from jax.experimental.pallas import tpu as pltpu


def _body(x_ref, kw_ref, kb_ref, qw_ref, qb_ref, vw_ref, vb_ref, rw_ref, rb_ref,
          o_ref, kk, qTT, vpp, *, c2, rc, n, ch, bi, bm):
    i = pl.program_id(1)

    @pl.when(i == 0)
    def _proj():
        w = jnp.concatenate([kw_ref[...], qw_ref[...], vw_ref[...]],
                            axis=1).astype(jnp.bfloat16)
        b = jnp.concatenate([kb_ref[...], qb_ref[...], vb_ref[...]], axis=1)
        for j in range(n // bm):
            xb = x_ref[0, j * bm:(j + 1) * bm, :].astype(jnp.bfloat16)
            kqv = jnp.dot(xb, w, preferred_element_type=jnp.float32) + b
            kk[j * bm:(j + 1) * bm, :] = (kqv[:, :rc] * c2).astype(jnp.bfloat16)
            kqvb = kqv.astype(jnp.bfloat16)
            qTT[:, j * bm:(j + 1) * bm] = kqvb[:, rc:2 * rc].T
            ones = jnp.ones((bm, rc), jnp.bfloat16)
            vpp[j * bm:(j + 1) * bm, :] = jnp.concatenate(
                [kqvb[:, 2 * rc:3 * rc], ones], axis=1)

    kh = kk[pl.ds(i * bi, bi), :]
    acc = jnp.zeros((bi, 2 * rc), jnp.float32)
    m_run = jnp.full((bi, 1), -jnp.inf, jnp.bfloat16)
    for c in range(n // ch):
        sc = jnp.dot(kh, qTT[:, c * ch:(c + 1) * ch],
                     preferred_element_type=jnp.float32).astype(jnp.bfloat16)
        m_new = jnp.maximum(m_run, jnp.max(sc, axis=-1, keepdims=True))
        e = jnp.exp2(sc - m_new)
        corr = jnp.exp2((m_run - m_new).astype(jnp.float32))
        pv = jnp.dot(e, vpp[c * ch:(c + 1) * ch, :],
                     preferred_element_type=jnp.float32)
        acc = acc * corr + pv
        m_run = m_new
    o = (acc[:, :rc] / acc[:, rc:]).astype(jnp.bfloat16)
    out = jnp.dot(o, rw_ref[...].astype(jnp.bfloat16),
                  preferred_element_type=jnp.float32) + rb_ref[...]
    o_ref[0] = out


def kernel(x, k_w, k_b, q_w, q_b, v_w, v_b, r_w, r_b):
    B, T, H, W, C = x.shape
    RC = k_w.shape[1]
    N = T * H * W
    c2 = math.log2(math.e) / math.sqrt(H * W * C)

    xf = x.reshape(B, N, C)
    BI = 1024 if N % 1024 == 0 else N
    CH = 512 if N % 512 == 0 else N
    BM = min(2048, N)

    out = pl.pallas_call(
        functools.partial(_body, c2=c2, rc=RC, n=N, ch=CH, bi=BI, bm=BM),
        grid=(B, N // BI),
        in_specs=[
            pl.BlockSpec((1, N, C), lambda b, i: (b, 0, 0)),
            pl.BlockSpec((C, RC), lambda b, i: (0, 0)),
            pl.BlockSpec((1, RC), lambda b, i: (0, 0)),
            pl.BlockSpec((C, RC), lambda b, i: (0, 0)),
            pl.BlockSpec((1, RC), lambda b, i: (0, 0)),
            pl.BlockSpec((C, RC), lambda b, i: (0, 0)),
            pl.BlockSpec((1, RC), lambda b, i: (0, 0)),
            pl.BlockSpec((RC, C), lambda b, i: (0, 0)),
            pl.BlockSpec((1, C), lambda b, i: (0, 0)),
        ],
        out_specs=pl.BlockSpec((1, BI, C), lambda b, i: (b, i, 0)),
        out_shape=jax.ShapeDtypeStruct((B, N, C), jnp.float32),
        scratch_shapes=[
            pltpu.VMEM((N, RC), jnp.bfloat16),
            pltpu.VMEM((RC, N), jnp.bfloat16),
            pltpu.VMEM((N, 2 * RC), jnp.bfloat16),
        ],
        compiler_params=pltpu.CompilerParams(
            dimension_semantics=("parallel", "arbitrary"),
            vmem_limit_bytes=56 * 1024 * 1024,
        ),
        name="a3d_fused",
    )(xf, k_w, k_b.reshape(1, RC), q_w, q_b.reshape(1, RC),
      v_w, v_b.reshape(1, RC), r_w, r_b.reshape(1, C))

    return out.reshape(B, T, H, W, C)


# speedup vs baseline: 1.0135x; 1.0135x over previous
import functools
import math

import jax
import jax.numpy as jnp
from jax.experimental import pallas as pl
from jax.experimental.pallas import tpu as pltpu


def _body(x_ref, kw_ref, kb_ref, qw_ref, qb_ref, vw_ref, vb_ref, rw_ref, rb_ref,
          o_ref, kk, qTT, vpp, *, c2, rc, n, ch, bi, bm):
    i = pl.program_id(1)

    @pl.when(i == 0)
    def _proj():
        w = jnp.concatenate([kw_ref[...], qw_ref[...], vw_ref[...]],
                            axis=1).astype(jnp.bfloat16)
        b = jnp.concatenate([kb_ref[...], qb_ref[...], vb_ref[...]], axis=1)
        for j in range(n // bm):
            xb = x_ref[0, j * bm:(j + 1) * bm, :].astype(jnp.bfloat16)
            kqv = jnp.dot(xb, w, preferred_element_type=jnp.float32) + b
            kqv = kqv.astype(jnp.bfloat16)
            kk[j * bm:(j + 1) * bm, :] = kqv[:, :rc]
            qTT[:, j * bm:(j + 1) * bm] = kqv[:, rc:2 * rc].T
            ones = jnp.ones((bm, rc), jnp.bfloat16)
            vpp[j * bm:(j + 1) * bm, :] = jnp.concatenate(
                [kqv[:, 2 * rc:3 * rc], ones], axis=1)

    kh = kk[pl.ds(i * bi, bi), :]
    acc = jnp.zeros((bi, 2 * rc), jnp.float32)
    m_run = jnp.full((bi, 1), -jnp.inf, jnp.bfloat16)
    for c in range(n // ch):
        sc = jnp.dot(kh, qTT[:, c * ch:(c + 1) * ch],
                     preferred_element_type=jnp.float32).astype(jnp.bfloat16)
        m_new = jnp.maximum(m_run, jnp.max(sc, axis=-1, keepdims=True))
        e = jnp.exp2((sc - m_new) * jnp.bfloat16(c2))
        corr = jnp.exp2((m_run - m_new).astype(jnp.float32) * c2)
        pv = jnp.dot(e, vpp[c * ch:(c + 1) * ch, :],
                     preferred_element_type=jnp.float32)
        acc = acc * corr + pv
        m_run = m_new
    o = (acc[:, :rc] / acc[:, rc:]).astype(jnp.bfloat16)
    out = jnp.dot(o, rw_ref[...].astype(jnp.bfloat16),
                  preferred_element_type=jnp.float32) + rb_ref[...]
    o_ref[0] = out


def kernel(x, k_w, k_b, q_w, q_b, v_w, v_b, r_w, r_b):
    B, T, H, W, C = x.shape
    RC = k_w.shape[1]
    N = T * H * W
    c2 = math.log2(math.e) / math.sqrt(H * W * C)

    xf = x.reshape(B, N, C)
    BI = 1024 if N % 1024 == 0 else N
    CH = 512 if N % 512 == 0 else N
    BM = min(4096, N)

    out = pl.pallas_call(
        functools.partial(_body, c2=c2, rc=RC, n=N, ch=CH, bi=BI, bm=BM),
        grid=(B, N // BI),
        in_specs=[
            pl.BlockSpec((1, N, C), lambda b, i: (b, 0, 0)),
            pl.BlockSpec((C, RC), lambda b, i: (0, 0)),
            pl.BlockSpec((1, RC), lambda b, i: (0, 0)),
            pl.BlockSpec((C, RC), lambda b, i: (0, 0)),
            pl.BlockSpec((1, RC), lambda b, i: (0, 0)),
            pl.BlockSpec((C, RC), lambda b, i: (0, 0)),
            pl.BlockSpec((1, RC), lambda b, i: (0, 0)),
            pl.BlockSpec((RC, C), lambda b, i: (0, 0)),
            pl.BlockSpec((1, C), lambda b, i: (0, 0)),
        ],
        out_specs=pl.BlockSpec((1, BI, C), lambda b, i: (b, i, 0)),
        out_shape=jax.ShapeDtypeStruct((B, N, C), jnp.float32),
        scratch_shapes=[
            pltpu.VMEM((N, RC), jnp.bfloat16),
            pltpu.VMEM((RC, N), jnp.bfloat16),
            pltpu.VMEM((N, 2 * RC), jnp.bfloat16),
        ],
        compiler_params=pltpu.CompilerParams(
            dimension_semantics=("parallel", "arbitrary"),
            vmem_limit_bytes=56 * 1024 * 1024,
        ),
        name="a3d_fused",
    )(xf, k_w, k_b.reshape(1, RC), q_w, q_b.reshape(1, RC),
      v_w, v_b.reshape(1, RC), r_w, r_b.reshape(1, C))

    return out.reshape(B, T, H, W, C)
